# baseline (speedup 1.0000x reference)
import math

import jax
import jax.numpy as jnp
from jax.experimental import pallas as pl
from jax.experimental.pallas import tpu as pltpu

_D = 128
_L = 64
_H = 2
_HS = 128
_DH = _H * _HS
_FF = 256
_EPS = 1e-6
_BT = 32
_NEG = -1e9
_BF16 = jnp.bfloat16


def _layernorm(y, gamma, beta, bd):
    n = y.shape[-1]
    yb = y.astype(_BF16)
    ycat = jnp.concatenate([yb, yb * yb], axis=-1)
    s12 = jnp.dot(ycat, bd, preferred_element_type=jnp.float32)
    s1 = s12[:, :n]
    s2 = s12[:, n:]
    mu = s1 * (1.0 / n)
    cen = y - mu
    var = jnp.maximum((s2 - s1 * mu) * (1.0 / (n - 1)), 1e-30)
    inv = jax.lax.rsqrt(var)
    return gamma * cen * inv + beta


def _masked_attention(q, k, v, xq2, wf, bf, madd):
    y = xq2 + bf
    for h in range(_H):
        sl = slice(h * _HS, (h + 1) * _HS)
        s = jax.lax.dot_general(q[..., sl], k[..., sl],
                                (((2,), (2,)), ((0,), (0,))),
                                preferred_element_type=jnp.float32)
        e = jnp.exp(s + madd)
        p = e * pl.reciprocal(jnp.sum(e, axis=-1, keepdims=True), approx=True)
        ctx = jax.lax.dot_general(p.astype(_BF16), v[..., sl],
                                  (((2,), (1,)), ((0,), (0,))),
                                  preferred_element_type=jnp.float32)
        y = y + jnp.dot(ctx.astype(_BF16).reshape(-1, _HS), wf[sl, :],
                        preferred_element_type=jnp.float32)
    return y


def _block_body(xt_ref, xo_ref, xc_ref, xr_ref, xs_ref, act_ref,
                wqkv_ref, bqkv_ref, wf_ref, bf_ref, ag_ref, ab_ref,
                w1_ref, b1_ref, w2_ref, b2_ref, fg_ref, fb_ref,
                ot_ref, oo_ref, oc_ref, or_ref):
    T = _BT * _L
    xt = xt_ref[...].reshape(T, _D)
    xo = xo_ref[...].reshape(T, _D)
    xc = xc_ref[...].reshape(T, _D)
    xr = xr_ref[...].reshape(T, _D)

    rr = jax.lax.broadcasted_iota(jnp.int32, (2 * _D, 2 * _D), 0)
    cc = jax.lax.broadcasted_iota(jnp.int32, (2 * _D, 2 * _D), 1)
    bd = ((rr < _D) == (cc < _D)).astype(_BF16)

    act = act_ref[...]
    row = jax.lax.broadcasted_iota(jnp.int32, (_BT, _L, _L), 1)
    col = jax.lax.broadcasted_iota(jnp.int32, (_BT, _L, _L), 2)
    live = act != 0.0
    madd = jnp.where(jnp.logical_and(live, col > row), _NEG, 0.0)
    qgate = act.astype(_BF16)

    def self_attn(x2, i):
        qkv = (jnp.dot(x2.astype(_BF16), wqkv_ref[i],
                       preferred_element_type=jnp.float32).astype(_BF16)
               + bqkv_ref[i]).reshape(_BT, _L, 3 * _DH)
        y = _masked_attention(qkv[..., :_DH] * qgate, qkv[..., _DH:2 * _DH],
                              qkv[..., 2 * _DH:], x2, wf_ref[i], bf_ref[i],
                              madd)
        return _layernorm(y, ag_ref[i], ab_ref[i], bd)

    def cross_attn(xq2, xkv2b, i):
        w = wqkv_ref[i]
        b = bqkv_ref[i]
        q = (jnp.dot(xq2.astype(_BF16), w[:, :_DH],
                     preferred_element_type=jnp.float32).astype(_BF16)
             + b[:, :_DH]).reshape(_BT, _L, _DH)
        kv = (jnp.dot(xkv2b, w[:, _DH:],
                      preferred_element_type=jnp.float32).astype(_BF16)
              + b[:, _DH:]).reshape(_BT, _L, 2 * _DH)
        y = _masked_attention(q * qgate, kv[..., :_DH], kv[..., _DH:], xq2,
                              wf_ref[i], bf_ref[i], madd)
        return _layernorm(y, ag_ref[i], ab_ref[i], bd)

    def ffn(x2, j):
        h = jnp.maximum(jnp.dot(x2.astype(_BF16), w1_ref[j],
                                preferred_element_type=jnp.float32
                                ).astype(_BF16) + b1_ref[j],
                        jnp.bfloat16(0.0))
        y = jnp.dot(h, w2_ref[j],
                    preferred_element_type=jnp.float32) + b2_ref[j] + x2
        return _layernorm(y, fg_ref[j], fb_ref[j], bd)

    c_sa = self_attn(xc, 0)
    r_sa = self_attn(xr, 1)
    t_sa = self_attn(xt, 2)
    o_sa = self_attn(xo, 3)
    t_ca = cross_attn(t_sa, xo.astype(_BF16), 4)
    o_ca = cross_attn(o_sa, xt.astype(_BF16), 5)

    xs = xs_ref[...].reshape(T, _D)
    ot_ref[...] = ffn(t_ca + xs, 0).reshape(_BT, _L, _D)
    oo_ref[...] = ffn(o_ca + xs, 1).reshape(_BT, _L, _D)
    oc_ref[...] = ffn(c_sa, 2).reshape(_BT, _L, _D)
    or_ref[...] = ffn(r_sa, 3).reshape(_BT, _L, _D)


def kernel(x_t, x_o, x_chemo_iv, x_radio_iv, x_s, active, sa_chemo_iv_wq, sa_chemo_iv_bq, sa_chemo_iv_wk, sa_chemo_iv_bk, sa_chemo_iv_wv, sa_chemo_iv_bv, sa_chemo_iv_wf, sa_chemo_iv_bf, sa_chemo_iv_gamma, sa_chemo_iv_beta, sa_radio_iv_wq, sa_radio_iv_bq, sa_radio_iv_wk, sa_radio_iv_bk, sa_radio_iv_wv, sa_radio_iv_bv, sa_radio_iv_wf, sa_radio_iv_bf, sa_radio_iv_gamma, sa_radio_iv_beta, sa_t_wq, sa_t_bq, sa_t_wk, sa_t_bk, sa_t_wv, sa_t_bv, sa_t_wf, sa_t_bf, sa_t_gamma, sa_t_beta, sa_o_wq, sa_o_bq, sa_o_wk, sa_o_bk, sa_o_wv, sa_o_bv, sa_o_wf, sa_o_bf, sa_o_gamma, sa_o_beta, ca_to_wq, ca_to_bq, ca_to_wk, ca_to_bk, ca_to_wv, ca_to_bv, ca_to_wf, ca_to_bf, ca_to_gamma, ca_to_beta, ca_ot_wq, ca_ot_bq, ca_ot_wk, ca_ot_bk, ca_ot_wv, ca_ot_bv, ca_ot_wf, ca_ot_bf, ca_ot_gamma, ca_ot_beta, ff_t_w1, ff_t_b1, ff_t_w2, ff_t_b2, ff_t_gamma, ff_t_beta, ff_o_w1, ff_o_b1, ff_o_w2, ff_o_b2, ff_o_gamma, ff_o_beta, ff_chemo_iv_w1, ff_chemo_iv_b1, ff_chemo_iv_w2, ff_chemo_iv_b2, ff_chemo_iv_gamma, ff_chemo_iv_beta, ff_radio_iv_w1, ff_radio_iv_b1, ff_radio_iv_w2, ff_radio_iv_b2, ff_radio_iv_gamma, ff_radio_iv_beta):
    B, L, D = x_t.shape
    nb = B // _BT
    inv_scale = 1.0 / math.sqrt(_HS)

    mha = [
        (sa_chemo_iv_wq, sa_chemo_iv_bq, sa_chemo_iv_wk, sa_chemo_iv_bk,
         sa_chemo_iv_wv, sa_chemo_iv_bv, sa_chemo_iv_wf, sa_chemo_iv_bf,
         sa_chemo_iv_gamma, sa_chemo_iv_beta),
        (sa_radio_iv_wq, sa_radio_iv_bq, sa_radio_iv_wk, sa_radio_iv_bk,
         sa_radio_iv_wv, sa_radio_iv_bv, sa_radio_iv_wf, sa_radio_iv_bf,
         sa_radio_iv_gamma, sa_radio_iv_beta),
        (sa_t_wq, sa_t_bq, sa_t_wk, sa_t_bk, sa_t_wv, sa_t_bv, sa_t_wf,
         sa_t_bf, sa_t_gamma, sa_t_beta),
        (sa_o_wq, sa_o_bq, sa_o_wk, sa_o_bk, sa_o_wv, sa_o_bv, sa_o_wf,
         sa_o_bf, sa_o_gamma, sa_o_beta),
        (ca_to_wq, ca_to_bq, ca_to_wk, ca_to_bk, ca_to_wv, ca_to_bv,
         ca_to_wf, ca_to_bf, ca_to_gamma, ca_to_beta),
        (ca_ot_wq, ca_ot_bq, ca_ot_wk, ca_ot_bk, ca_ot_wv, ca_ot_bv,
         ca_ot_wf, ca_ot_bf, ca_ot_gamma, ca_ot_beta),
    ]
    ffn = [
        (ff_t_w1, ff_t_b1, ff_t_w2, ff_t_b2, ff_t_gamma, ff_t_beta),
        (ff_o_w1, ff_o_b1, ff_o_w2, ff_o_b2, ff_o_gamma, ff_o_beta),
        (ff_chemo_iv_w1, ff_chemo_iv_b1, ff_chemo_iv_w2, ff_chemo_iv_b2,
         ff_chemo_iv_gamma, ff_chemo_iv_beta),
        (ff_radio_iv_w1, ff_radio_iv_b1, ff_radio_iv_w2, ff_radio_iv_b2,
         ff_radio_iv_gamma, ff_radio_iv_beta),
    ]

    wqkv = jnp.stack([jnp.concatenate([p[0] * inv_scale, p[2], p[4]], axis=1)
                      for p in mha]).astype(_BF16)
    bqkv = jnp.stack([jnp.concatenate([p[1] * inv_scale, p[3], p[5]])[None, :]
                      for p in mha]).astype(_BF16)
    wf = jnp.stack([p[6] for p in mha]).astype(_BF16)
    bf = jnp.stack([p[7][None, :] for p in mha])
    ag = jnp.stack([p[8][None, :] for p in mha])
    ab = jnp.stack([p[9][None, :] for p in mha])

    w1 = jnp.stack([p[0] for p in ffn]).astype(_BF16)
    b1 = jnp.stack([p[1][None, :] for p in ffn]).astype(_BF16)
    w2 = jnp.stack([p[2] for p in ffn]).astype(_BF16)
    b2 = jnp.stack([p[3][None, :] for p in ffn])
    fg = jnp.stack([p[4][None, :] for p in ffn])
    fb = jnp.stack([p[5][None, :] for p in ffn])

    act = active.astype(jnp.float32).reshape(B, L, 1)

    tok_spec = pl.BlockSpec((_BT, L, D), lambda i: (i, 0, 0))
    act_spec = pl.BlockSpec((_BT, L, 1), lambda i: (i, 0, 0))

    def _w_spec(a):
        return pl.BlockSpec(a.shape, lambda i: (0, 0, 0))

    in_specs = [tok_spec] * 5 + [act_spec] + [
        _w_spec(wqkv), _w_spec(bqkv), _w_spec(wf), _w_spec(bf),
        _w_spec(ag), _w_spec(ab),
        _w_spec(w1), _w_spec(b1), _w_spec(w2), _w_spec(b2),
        _w_spec(fg), _w_spec(fb),
    ]
    out_sds = jax.ShapeDtypeStruct((B, L, D), x_t.dtype)

    flops_attn = 6 * B * (2 * L * D * 3 * _DH + _H * 4 * L * L * _HS
                          + 2 * L * _DH * D)
    flops_ffn = 4 * B * 4 * L * D * _FF
    cost = pl.CostEstimate(
        flops=flops_attn + flops_ffn,
        transcendentals=6 * B * (_H * L * L + 2 * L) + 4 * B * 2 * L,
        bytes_accessed=9 * B * L * D * 4 + B * L * 4,
    )

    outs = pl.pallas_call(
        _block_body,
        out_shape=(out_sds, out_sds, out_sds, out_sds),
        grid=(nb,),
        in_specs=in_specs,
        out_specs=(tok_spec, tok_spec, tok_spec, tok_spec),
        compiler_params=pltpu.CompilerParams(
            dimension_semantics=("parallel",),
            vmem_limit_bytes=56 * 1024 * 1024),
        cost_estimate=cost,
    )(x_t, x_o, x_chemo_iv, x_radio_iv, x_s, act,
      wqkv, bqkv, wf, bf, ag, ab, w1, b1, w2, b2, fg, fb)
    return outs


# speedup vs baseline: 1.0005x; 1.0005x over previous
import math

import jax
import jax.numpy as jnp
from jax.experimental import pallas as pl
from jax.experimental.pallas import tpu as pltpu

_D = 128
_L = 64
_H = 2
_HS = 128
_DH = _H * _HS
_FF = 256
_EPS = 1e-6
_BT = 32
_NEG = -1e9
_BF16 = jnp.bfloat16


def _layernorm(y, gamma, beta, bd):
    n = y.shape[-1]
    yb = y.astype(_BF16)
    ycat = jnp.concatenate([yb, yb * yb], axis=-1)
    s12 = jnp.dot(ycat, bd, preferred_element_type=jnp.float32)
    s1 = s12[:, :n]
    s2 = s12[:, n:]
    mu = s1 * (1.0 / n)
    cen = y - mu
    var = jnp.maximum((s2 - s1 * mu) * (1.0 / (n - 1)), 1e-30)
    inv = jax.lax.rsqrt(var)
    return gamma * cen * inv + beta


def _masked_attention(q, k, v, xq2, wf, bf, madd):
    ctxs = []
    for h in range(_H):
        sl = slice(h * _HS, (h + 1) * _HS)
        s = jax.lax.dot_general(q[..., sl], k[..., sl],
                                (((2,), (2,)), ((0,), (0,))),
                                preferred_element_type=jnp.float32)
        e = jnp.exp(s + madd)
        p = e * pl.reciprocal(jnp.sum(e, axis=-1, keepdims=True), approx=True)
        ctx = jax.lax.dot_general(p.astype(_BF16), v[..., sl],
                                  (((2,), (1,)), ((0,), (0,))),
                                  preferred_element_type=jnp.float32)
        ctxs.append(ctx.astype(_BF16))
    ctx2 = jnp.concatenate(ctxs, axis=-1).reshape(-1, _DH)
    return xq2 + bf + jnp.dot(ctx2, wf, preferred_element_type=jnp.float32)


def _block_body(xt_ref, xo_ref, xc_ref, xr_ref, xs_ref, act_ref,
                wk1_ref, bb_ref, wk2_ref, pf_ref,
                ot_ref, oo_ref, oc_ref, or_ref):
    T = _BT * _L
    xt = xt_ref[...].reshape(T, _D)
    xo = xo_ref[...].reshape(T, _D)
    xc = xc_ref[...].reshape(T, _D)
    xr = xr_ref[...].reshape(T, _D)

    rr = jax.lax.broadcasted_iota(jnp.int32, (2 * _D, 2 * _D), 0)
    cc = jax.lax.broadcasted_iota(jnp.int32, (2 * _D, 2 * _D), 1)
    bd = ((rr < _D) == (cc < _D)).astype(_BF16)

    act = act_ref[...]
    row = jax.lax.broadcasted_iota(jnp.int32, (_BT, _L, _L), 1)
    col = jax.lax.broadcasted_iota(jnp.int32, (_BT, _L, _L), 2)
    live = act != 0.0
    madd = jnp.where(jnp.logical_and(live, col > row), _NEG, 0.0)
    qgate = act.astype(_BF16)

    def _attn_w(i):
        return wk1_ref[:, i * 3 * _DH:(i + 1) * 3 * _DH]

    def _attn_b(i):
        return bb_ref[:, i * 3 * _DH:(i + 1) * 3 * _DH]

    def _f32p(idx):
        return pf_ref[:, idx * _D:(idx + 1) * _D]

    def self_attn(x2, i):
        qkv = (jnp.dot(x2.astype(_BF16), _attn_w(i),
                       preferred_element_type=jnp.float32).astype(_BF16)
               + _attn_b(i)).reshape(_BT, _L, 3 * _DH)
        y = _masked_attention(qkv[..., :_DH] * qgate, qkv[..., _DH:2 * _DH],
                              qkv[..., 2 * _DH:], x2,
                              wk2_ref[:, i * _D:(i + 1) * _D], _f32p(i), madd)
        return _layernorm(y, _f32p(6 + i), _f32p(12 + i), bd)

    def cross_attn(xq2, xkv2b, i):
        w = _attn_w(i)
        b = _attn_b(i)
        q = (jnp.dot(xq2.astype(_BF16), w[:, :_DH],
                     preferred_element_type=jnp.float32).astype(_BF16)
             + b[:, :_DH]).reshape(_BT, _L, _DH)
        kv = (jnp.dot(xkv2b, w[:, _DH:],
                      preferred_element_type=jnp.float32).astype(_BF16)
              + b[:, _DH:]).reshape(_BT, _L, 2 * _DH)
        y = _masked_attention(q * qgate, kv[..., :_DH], kv[..., _DH:], xq2,
                              wk2_ref[:, i * _D:(i + 1) * _D], _f32p(i), madd)
        return _layernorm(y, _f32p(6 + i), _f32p(12 + i), bd)

    _W1OFF = 6 * 3 * _DH

    def ffn(x2, j):
        h = jnp.maximum(
            jnp.dot(x2.astype(_BF16),
                    wk1_ref[:, _W1OFF + j * _FF:_W1OFF + (j + 1) * _FF],
                    preferred_element_type=jnp.float32).astype(_BF16)
            + bb_ref[:, _W1OFF + j * _FF:_W1OFF + (j + 1) * _FF],
            jnp.bfloat16(0.0))
        y = (jnp.dot(h, wk2_ref[:, (6 + j) * _D:(7 + j) * _D],
                     preferred_element_type=jnp.float32)
             + _f32p(18 + j) + x2)
        return _layernorm(y, _f32p(22 + j), _f32p(26 + j), bd)

    c_sa = self_attn(xc, 0)
    r_sa = self_attn(xr, 1)
    t_sa = self_attn(xt, 2)
    o_sa = self_attn(xo, 3)
    t_ca = cross_attn(t_sa, xo.astype(_BF16), 4)
    o_ca = cross_attn(o_sa, xt.astype(_BF16), 5)

    xs = xs_ref[...].reshape(T, _D)
    ot_ref[...] = ffn(t_ca + xs, 0).reshape(_BT, _L, _D)
    oo_ref[...] = ffn(o_ca + xs, 1).reshape(_BT, _L, _D)
    oc_ref[...] = ffn(c_sa, 2).reshape(_BT, _L, _D)
    or_ref[...] = ffn(r_sa, 3).reshape(_BT, _L, _D)


def kernel(x_t, x_o, x_chemo_iv, x_radio_iv, x_s, active, sa_chemo_iv_wq, sa_chemo_iv_bq, sa_chemo_iv_wk, sa_chemo_iv_bk, sa_chemo_iv_wv, sa_chemo_iv_bv, sa_chemo_iv_wf, sa_chemo_iv_bf, sa_chemo_iv_gamma, sa_chemo_iv_beta, sa_radio_iv_wq, sa_radio_iv_bq, sa_radio_iv_wk, sa_radio_iv_bk, sa_radio_iv_wv, sa_radio_iv_bv, sa_radio_iv_wf, sa_radio_iv_bf, sa_radio_iv_gamma, sa_radio_iv_beta, sa_t_wq, sa_t_bq, sa_t_wk, sa_t_bk, sa_t_wv, sa_t_bv, sa_t_wf, sa_t_bf, sa_t_gamma, sa_t_beta, sa_o_wq, sa_o_bq, sa_o_wk, sa_o_bk, sa_o_wv, sa_o_bv, sa_o_wf, sa_o_bf, sa_o_gamma, sa_o_beta, ca_to_wq, ca_to_bq, ca_to_wk, ca_to_bk, ca_to_wv, ca_to_bv, ca_to_wf, ca_to_bf, ca_to_gamma, ca_to_beta, ca_ot_wq, ca_ot_bq, ca_ot_wk, ca_ot_bk, ca_ot_wv, ca_ot_bv, ca_ot_wf, ca_ot_bf, ca_ot_gamma, ca_ot_beta, ff_t_w1, ff_t_b1, ff_t_w2, ff_t_b2, ff_t_gamma, ff_t_beta, ff_o_w1, ff_o_b1, ff_o_w2, ff_o_b2, ff_o_gamma, ff_o_beta, ff_chemo_iv_w1, ff_chemo_iv_b1, ff_chemo_iv_w2, ff_chemo_iv_b2, ff_chemo_iv_gamma, ff_chemo_iv_beta, ff_radio_iv_w1, ff_radio_iv_b1, ff_radio_iv_w2, ff_radio_iv_b2, ff_radio_iv_gamma, ff_radio_iv_beta):
    B, L, D = x_t.shape
    nb = B // _BT
    inv_scale = 1.0 / math.sqrt(_HS)

    mha = [
        (sa_chemo_iv_wq, sa_chemo_iv_bq, sa_chemo_iv_wk, sa_chemo_iv_bk,
         sa_chemo_iv_wv, sa_chemo_iv_bv, sa_chemo_iv_wf, sa_chemo_iv_bf,
         sa_chemo_iv_gamma, sa_chemo_iv_beta),
        (sa_radio_iv_wq, sa_radio_iv_bq, sa_radio_iv_wk, sa_radio_iv_bk,
         sa_radio_iv_wv, sa_radio_iv_bv, sa_radio_iv_wf, sa_radio_iv_bf,
         sa_radio_iv_gamma, sa_radio_iv_beta),
        (sa_t_wq, sa_t_bq, sa_t_wk, sa_t_bk, sa_t_wv, sa_t_bv, sa_t_wf,
         sa_t_bf, sa_t_gamma, sa_t_beta),
        (sa_o_wq, sa_o_bq, sa_o_wk, sa_o_bk, sa_o_wv, sa_o_bv, sa_o_wf,
         sa_o_bf, sa_o_gamma, sa_o_beta),
        (ca_to_wq, ca_to_bq, ca_to_wk, ca_to_bk, ca_to_wv, ca_to_bv,
         ca_to_wf, ca_to_bf, ca_to_gamma, ca_to_beta),
        (ca_ot_wq, ca_ot_bq, ca_ot_wk, ca_ot_bk, ca_ot_wv, ca_ot_bv,
         ca_ot_wf, ca_ot_bf, ca_ot_gamma, ca_ot_beta),
    ]
    ffn = [
        (ff_t_w1, ff_t_b1, ff_t_w2, ff_t_b2, ff_t_gamma, ff_t_beta),
        (ff_o_w1, ff_o_b1, ff_o_w2, ff_o_b2, ff_o_gamma, ff_o_beta),
        (ff_chemo_iv_w1, ff_chemo_iv_b1, ff_chemo_iv_w2, ff_chemo_iv_b2,
         ff_chemo_iv_gamma, ff_chemo_iv_beta),
        (ff_radio_iv_w1, ff_radio_iv_b1, ff_radio_iv_w2, ff_radio_iv_b2,
         ff_radio_iv_gamma, ff_radio_iv_beta),
    ]

    wk1 = jnp.concatenate(
        [jnp.concatenate([p[0] * inv_scale, p[2], p[4]], axis=1) for p in mha]
        + [p[0] for p in ffn], axis=1).astype(_BF16)
    bb = jnp.concatenate(
        [jnp.concatenate([p[1] * inv_scale, p[3], p[5]]) for p in mha]
        + [p[1] for p in ffn])[None, :].astype(_BF16)
    wk2 = jnp.concatenate([p[6] for p in mha] + [p[2] for p in ffn],
                          axis=1).astype(_BF16)
    pf = jnp.concatenate(
        [p[7] for p in mha] + [p[8] for p in mha] + [p[9] for p in mha]
        + [p[3] for p in ffn] + [p[4] for p in ffn]
        + [p[5] for p in ffn])[None, :]

    act = active.astype(jnp.float32).reshape(B, L, 1)

    tok_spec = pl.BlockSpec((_BT, L, D), lambda i: (i, 0, 0))
    act_spec = pl.BlockSpec((_BT, L, 1), lambda i: (i, 0, 0))

    def _w_spec(a):
        return pl.BlockSpec(a.shape, lambda i: (0, 0))

    in_specs = [tok_spec] * 5 + [act_spec] + [
        _w_spec(wk1), _w_spec(bb), _w_spec(wk2), _w_spec(pf),
    ]
    out_sds = jax.ShapeDtypeStruct((B, L, D), x_t.dtype)

    flops_attn = 6 * B * (2 * L * D * 3 * _DH + _H * 4 * L * L * _HS
                          + 2 * L * _DH * D)
    flops_ffn = 4 * B * 4 * L * D * _FF
    cost = pl.CostEstimate(
        flops=flops_attn + flops_ffn,
        transcendentals=6 * B * (_H * L * L + 2 * L) + 4 * B * 2 * L,
        bytes_accessed=9 * B * L * D * 4 + B * L * 4,
    )

    outs = pl.pallas_call(
        _block_body,
        out_shape=(out_sds, out_sds, out_sds, out_sds),
        grid=(nb,),
        in_specs=in_specs,
        out_specs=(tok_spec, tok_spec, tok_spec, tok_spec),
        compiler_params=pltpu.CompilerParams(
            dimension_semantics=("parallel",),
            vmem_limit_bytes=56 * 1024 * 1024),
        cost_estimate=cost,
    )(x_t, x_o, x_chemo_iv, x_radio_iv, x_s, act, wk1, bb, wk2, pf)
    return outs
